# Initial kernel scaffold; baseline (speedup 1.0000x reference)
import jax
import jax.numpy as jnp
from jax.experimental import pallas as pl


def kernel(input_ids, token_type_ids, match_entity, sf_entity, match_token, sf_token, etype_ids, word_emb, token_type_emb, pos_emb, match_entity_emb, sf_entity_emb, match_token_emb, sf_token_emb, etype_emb, gamma, beta):
    raise NotImplementedError("write your pallas kernel here")


# speedup vs baseline: 4.5186x; 4.5186x over previous
import functools

import jax
import jax.numpy as jnp
from jax import lax
from jax.experimental import pallas as pl
from jax.experimental.pallas import tpu as pltpu
from jax.experimental.pallas import tpu_sc as plsc

VOCAB = 30522
HIDDEN = 768
MAX_POS = 2048
SF_LEVEL = 8
N_ETYPE = 16
B, S = 4, 2048
EPS = 1e-12

NTOK = B * S


_NC = 2
_NS = 16
_NW = _NC * _NS
_ROWS_PER_W = NTOK // _NW
_CHUNK = 64
_NCH = _ROWS_PER_W // _CHUNK


@functools.cache
def _make_sc_gather():
    mesh = plsc.VectorSubcoreMesh(core_axis_name="c", subcore_axis_name="s")

    @functools.partial(
        pl.kernel,
        mesh=mesh,
        out_type=jax.ShapeDtypeStruct((NTOK, HIDDEN), jnp.float32),
        scratch_types=[
            pltpu.VMEM((_ROWS_PER_W,), jnp.int32),
            pltpu.VMEM((2, _CHUNK, HIDDEN), jnp.float32),
            pltpu.SemaphoreType.DMA,
            pltpu.SemaphoreType.DMA,
            pltpu.SemaphoreType.DMA,
            pltpu.SemaphoreType.DMA,
        ],
    )
    def _sc_gather(ids_hbm, table_hbm, out_hbm, idx_v, rows_v, g0, g1, w0, w1):
        wid = lax.axis_index("s") * _NC + lax.axis_index("c")
        base = wid * _ROWS_PER_W
        pltpu.sync_copy(ids_hbm.at[pl.ds(base, _ROWS_PER_W)], idx_v)

        gsems = (g0, g1)
        wsems = (w0, w1)
        gathers = [None] * _NCH
        writes = [None] * _NCH

        def _issue_gather(ci):
            return pltpu.async_copy(
                table_hbm.at[idx_v.at[pl.ds(ci * _CHUNK, _CHUNK)]],
                rows_v.at[ci % 2],
                gsems[ci % 2],
            )

        gathers[0] = _issue_gather(0)
        for ci in range(_NCH):
            if ci + 1 < _NCH:
                if ci - 1 >= 0:
                    writes[ci - 1].wait()
                gathers[ci + 1] = _issue_gather(ci + 1)
            gathers[ci].wait()
            writes[ci] = pltpu.async_copy(
                rows_v.at[ci % 2],
                out_hbm.at[pl.ds(base + ci * _CHUNK, _CHUNK)],
                wsems[ci % 2],
            )
        writes[_NCH - 2].wait()
        writes[_NCH - 1].wait()

    return _sc_gather


_T = 512
_NBLK = NTOK // _T
_SBLK = S // _T
_NSMALL = 64

_OFF_TT = 0
_OFF_ME = 2
_OFF_MT = 4
_OFF_SFE = 6
_OFF_SFT = 14
_OFF_ET = 22


def _tc_body(g_ref, p_ref, tt_ref, me_ref, mt_ref, sfe_ref, sft_ref, et_ref,
             small_ref, gamma_ref, beta_ref, out_ref):
    x = g_ref[...] + p_ref[...]

    col = lax.broadcasted_iota(jnp.int32, (_T, _NSMALL), 1)

    def onehot(idx_ref, off):
        idx = idx_ref[0, 0, :].reshape(_T, 1)
        return (col == idx + off).astype(jnp.float32)

    tt = (tt_ref[0, 0, :] > 0).astype(jnp.int32).reshape(_T, 1)
    oh = (col == tt + _OFF_TT).astype(jnp.float32)
    oh += onehot(me_ref, _OFF_ME)
    oh += onehot(mt_ref, _OFF_MT)
    oh += onehot(sfe_ref, _OFF_SFE)
    oh += onehot(sft_ref, _OFF_SFT)
    oh += onehot(et_ref, _OFF_ET)

    aux = lax.dot_general(
        oh, small_ref[...], (((1,), (0,)), ((), ())),
        preferred_element_type=jnp.float32,
        precision=lax.Precision.HIGHEST,
    )
    x = x + aux

    mu = jnp.mean(x, axis=-1, keepdims=True)
    xc = x - mu
    var = jnp.mean(xc * xc, axis=-1, keepdims=True)
    y = xc * lax.rsqrt(var + EPS)
    out_ref[...] = y * gamma_ref[...] + beta_ref[...]


def _tc_call(gathered, pos_emb, idxs, small, gamma2d, beta2d):
    idx_spec = pl.BlockSpec((1, 1, _T), lambda i: (i, 0, 0))
    return pl.pallas_call(
        _tc_body,
        grid=(_NBLK,),
        in_specs=[
            pl.BlockSpec((_T, HIDDEN), lambda i: (i, 0)),
            pl.BlockSpec((_T, HIDDEN), lambda i: (i % _SBLK, 0)),
            idx_spec, idx_spec, idx_spec, idx_spec, idx_spec, idx_spec,
            pl.BlockSpec((_NSMALL, HIDDEN), lambda i: (0, 0)),
            pl.BlockSpec((1, HIDDEN), lambda i: (0, 0)),
            pl.BlockSpec((1, HIDDEN), lambda i: (0, 0)),
        ],
        out_specs=pl.BlockSpec((_T, HIDDEN), lambda i: (i, 0)),
        out_shape=jax.ShapeDtypeStruct((NTOK, HIDDEN), jnp.float32),
    )(gathered, pos_emb, *idxs, small, gamma2d, beta2d)


def kernel(input_ids, token_type_ids, match_entity, sf_entity, match_token,
           sf_token, etype_ids, word_emb, token_type_emb, pos_emb,
           match_entity_emb, sf_entity_emb, match_token_emb, sf_token_emb,
           etype_emb, gamma, beta):
    ids = input_ids.reshape(NTOK).astype(jnp.int32)
    gathered = _make_sc_gather()(ids, word_emb)

    def prep(a):
        return a.reshape(_NBLK, 1, _T).astype(jnp.int32)

    idxs = (prep(token_type_ids), prep(match_entity), prep(match_token),
            prep(sf_entity), prep(sf_token), prep(etype_ids))

    small = jnp.zeros((_NSMALL, HIDDEN), jnp.float32)
    small = small.at[_OFF_TT:_OFF_TT + 2].set(token_type_emb)
    small = small.at[_OFF_ME:_OFF_ME + 2].set(match_entity_emb)
    small = small.at[_OFF_MT:_OFF_MT + 2].set(match_token_emb)
    small = small.at[_OFF_SFE:_OFF_SFE + SF_LEVEL].set(sf_entity_emb)
    small = small.at[_OFF_SFT:_OFF_SFT + SF_LEVEL].set(sf_token_emb)
    small = small.at[_OFF_ET:_OFF_ET + N_ETYPE].set(etype_emb)

    out = _tc_call(gathered, pos_emb, idxs, small,
                   gamma.reshape(1, HIDDEN), beta.reshape(1, HIDDEN))
    return out.reshape(B, S, HIDDEN)


# speedup vs baseline: 4.6219x; 1.0229x over previous
import functools

import jax
import jax.numpy as jnp
from jax import lax
from jax.experimental import pallas as pl
from jax.experimental.pallas import tpu as pltpu
from jax.experimental.pallas import tpu_sc as plsc

VOCAB = 30522
HIDDEN = 768
MAX_POS = 2048
SF_LEVEL = 8
N_ETYPE = 16
B, S = 4, 2048
EPS = 1e-12

NTOK = B * S


_NC = 2
_NS = 16
_NW = _NC * _NS
_ROWS_PER_W = NTOK // _NW
_CHUNK = 64
_NCH = _ROWS_PER_W // _CHUNK


@functools.cache
def _make_sc_gather():
    mesh = plsc.VectorSubcoreMesh(core_axis_name="c", subcore_axis_name="s")

    @functools.partial(
        pl.kernel,
        mesh=mesh,
        out_type=jax.ShapeDtypeStruct((NTOK, HIDDEN), jnp.float32),
        scratch_types=[
            pltpu.VMEM((_ROWS_PER_W,), jnp.int32),
            pltpu.VMEM((2, _CHUNK, HIDDEN), jnp.float32),
            pltpu.SemaphoreType.DMA,
            pltpu.SemaphoreType.DMA,
            pltpu.SemaphoreType.DMA,
            pltpu.SemaphoreType.DMA,
        ],
    )
    def _sc_gather(ids_hbm, table_hbm, out_hbm, idx_v, rows_v, g0, g1, w0, w1):
        wid = lax.axis_index("s") * _NC + lax.axis_index("c")
        base = wid * _ROWS_PER_W
        pltpu.sync_copy(ids_hbm.at[pl.ds(base, _ROWS_PER_W)], idx_v)

        gsems = (g0, g1)
        wsems = (w0, w1)
        gathers = [None] * _NCH
        writes = [None] * _NCH

        def _issue_gather(ci):
            return pltpu.async_copy(
                table_hbm.at[idx_v.at[pl.ds(ci * _CHUNK, _CHUNK)]],
                rows_v.at[ci % 2],
                gsems[ci % 2],
            )

        gathers[0] = _issue_gather(0)
        for ci in range(_NCH):
            if ci + 1 < _NCH:
                if ci - 1 >= 0:
                    writes[ci - 1].wait()
                gathers[ci + 1] = _issue_gather(ci + 1)
            gathers[ci].wait()
            writes[ci] = pltpu.async_copy(
                rows_v.at[ci % 2],
                out_hbm.at[pl.ds(base + ci * _CHUNK, _CHUNK)],
                wsems[ci % 2],
            )
        writes[_NCH - 2].wait()
        writes[_NCH - 1].wait()

    return _sc_gather


_T = 512
_NBLK = NTOK // _T
_SBLK = S // _T
_NSMALL = 64

_OFF_TT = 0
_OFF_ME = 2
_OFF_MT = 4
_OFF_SFE = 6
_OFF_SFT = 14
_OFF_ET = 22


def _tc_body(g_ref, p_ref, tt_ref, me_ref, mt_ref, sfe_ref, sft_ref, et_ref,
             small_ref, gamma_ref, beta_ref, out_ref):
    x = g_ref[...] + p_ref[...]

    col = lax.broadcasted_iota(jnp.int32, (_T, _NSMALL), 1)

    def onehot(idx_ref, off):
        idx = idx_ref[0, 0, :].reshape(_T, 1)
        return (col == idx + off).astype(jnp.float32)

    tt = (tt_ref[0, 0, :] > 0).astype(jnp.int32).reshape(_T, 1)
    oh = (col == tt + _OFF_TT).astype(jnp.float32)
    oh += onehot(me_ref, _OFF_ME)
    oh += onehot(mt_ref, _OFF_MT)
    oh += onehot(sfe_ref, _OFF_SFE)
    oh += onehot(sft_ref, _OFF_SFT)
    oh += onehot(et_ref, _OFF_ET)

    aux = lax.dot_general(
        oh, small_ref[...], (((1,), (0,)), ((), ())),
        preferred_element_type=jnp.float32,
        precision=lax.Precision.HIGHEST,
    )
    x = x + aux

    mu = jnp.mean(x, axis=-1, keepdims=True)
    xc = x - mu
    var = jnp.mean(xc * xc, axis=-1, keepdims=True)
    y = xc * lax.rsqrt(var + EPS)
    out_ref[...] = y * gamma_ref[...] + beta_ref[...]


def _tc_call(gathered, pos_emb, idxs, small, gamma2d, beta2d):
    tok = lambda sb, b: b * _SBLK + sb
    idx_spec = pl.BlockSpec((1, 1, _T), lambda sb, b: (tok(sb, b), 0, 0))
    return pl.pallas_call(
        _tc_body,
        grid=(_SBLK, _NBLK // _SBLK),
        in_specs=[
            pl.BlockSpec((_T, HIDDEN), lambda sb, b: (tok(sb, b), 0)),
            pl.BlockSpec((_T, HIDDEN), lambda sb, b: (sb, 0)),
            idx_spec, idx_spec, idx_spec, idx_spec, idx_spec, idx_spec,
            pl.BlockSpec((_NSMALL, HIDDEN), lambda sb, b: (0, 0)),
            pl.BlockSpec((1, HIDDEN), lambda sb, b: (0, 0)),
            pl.BlockSpec((1, HIDDEN), lambda sb, b: (0, 0)),
        ],
        out_specs=pl.BlockSpec((_T, HIDDEN), lambda sb, b: (tok(sb, b), 0)),
        out_shape=jax.ShapeDtypeStruct((NTOK, HIDDEN), jnp.float32),
    )(gathered, pos_emb, *idxs, small, gamma2d, beta2d)


def kernel(input_ids, token_type_ids, match_entity, sf_entity, match_token,
           sf_token, etype_ids, word_emb, token_type_emb, pos_emb,
           match_entity_emb, sf_entity_emb, match_token_emb, sf_token_emb,
           etype_emb, gamma, beta):
    ids = input_ids.reshape(NTOK).astype(jnp.int32)
    gathered = _make_sc_gather()(ids, word_emb)

    def prep(a):
        return a.reshape(_NBLK, 1, _T).astype(jnp.int32)

    idxs = (prep(token_type_ids), prep(match_entity), prep(match_token),
            prep(sf_entity), prep(sf_token), prep(etype_ids))

    small = jnp.zeros((_NSMALL, HIDDEN), jnp.float32)
    small = small.at[_OFF_TT:_OFF_TT + 2].set(token_type_emb)
    small = small.at[_OFF_ME:_OFF_ME + 2].set(match_entity_emb)
    small = small.at[_OFF_MT:_OFF_MT + 2].set(match_token_emb)
    small = small.at[_OFF_SFE:_OFF_SFE + SF_LEVEL].set(sf_entity_emb)
    small = small.at[_OFF_SFT:_OFF_SFT + SF_LEVEL].set(sf_token_emb)
    small = small.at[_OFF_ET:_OFF_ET + N_ETYPE].set(etype_emb)

    out = _tc_call(gathered, pos_emb, idxs, small,
                   gamma.reshape(1, HIDDEN), beta.reshape(1, HIDDEN))
    return out.reshape(B, S, HIDDEN)


# speedup vs baseline: 5.3075x; 1.1483x over previous
import functools

import jax
import jax.numpy as jnp
from jax import lax
from jax.experimental import pallas as pl
from jax.experimental.pallas import tpu as pltpu
from jax.experimental.pallas import tpu_sc as plsc

VOCAB = 30522
HIDDEN = 768
MAX_POS = 2048
SF_LEVEL = 8
N_ETYPE = 16
B, S = 4, 2048
EPS = 1e-12

NTOK = B * S


_NC = 2
_NS = 16
_NW = _NC * _NS
_ROWS_PER_W = NTOK // _NW
_CHUNK = 64
_NCH = _ROWS_PER_W // _CHUNK


@functools.cache
def _make_sc_gather():
    mesh = plsc.VectorSubcoreMesh(core_axis_name="c", subcore_axis_name="s")

    @functools.partial(
        pl.kernel,
        mesh=mesh,
        out_type=jax.ShapeDtypeStruct((NTOK, HIDDEN), jnp.float32),
        scratch_types=[
            pltpu.VMEM((_ROWS_PER_W,), jnp.int32),
            pltpu.VMEM((2, _CHUNK, HIDDEN), jnp.float32),
            pltpu.SemaphoreType.DMA,
            pltpu.SemaphoreType.DMA,
            pltpu.SemaphoreType.DMA,
            pltpu.SemaphoreType.DMA,
        ],
    )
    def _sc_gather(ids_hbm, table_hbm, out_hbm, idx_v, rows_v, g0, g1, w0, w1):
        wid = lax.axis_index("s") * _NC + lax.axis_index("c")
        base = wid * _ROWS_PER_W
        pltpu.sync_copy(ids_hbm.at[pl.ds(base, _ROWS_PER_W)], idx_v)

        gsems = (g0, g1)
        wsems = (w0, w1)
        gathers = [None] * _NCH
        writes = [None] * _NCH

        def _issue_gather(ci):
            return pltpu.async_copy(
                table_hbm.at[idx_v.at[pl.ds(ci * _CHUNK, _CHUNK)]],
                rows_v.at[ci % 2],
                gsems[ci % 2],
            )

        gathers[0] = _issue_gather(0)
        for ci in range(_NCH):
            if ci + 1 < _NCH:
                if ci - 1 >= 0:
                    writes[ci - 1].wait()
                gathers[ci + 1] = _issue_gather(ci + 1)
            gathers[ci].wait()
            writes[ci] = pltpu.async_copy(
                rows_v.at[ci % 2],
                out_hbm.at[pl.ds(base + ci * _CHUNK, _CHUNK)],
                wsems[ci % 2],
            )
        writes[_NCH - 2].wait()
        writes[_NCH - 1].wait()

    return _sc_gather


_T = 512
_NBLK = NTOK // _T
_SBLK = S // _T
_NSMALL = 64

_OFF_TT = 0
_OFF_ME = 2
_OFF_MT = 4
_OFF_SFE = 6
_OFF_SFT = 14
_OFF_ET = 22


def _tc_body(g_ref, p_ref, tt_ref, me_ref, mt_ref, sfe_ref, sft_ref, et_ref,
             hi_ref, lo_ref, gamma_ref, beta_ref, out_ref):
    x = g_ref[...] + p_ref[...]

    row = lax.broadcasted_iota(jnp.int32, (_NSMALL, _T), 0)

    def onehot_t(idx_ref, off):
        idx = idx_ref[0, :, :]
        return row == idx + off

    tt = (tt_ref[0, :, :] > 0).astype(jnp.int32)
    oh = (row == tt + _OFF_TT)
    oh |= onehot_t(me_ref, _OFF_ME)
    oh |= onehot_t(mt_ref, _OFF_MT)
    oh |= onehot_t(sfe_ref, _OFF_SFE)
    oh |= onehot_t(sft_ref, _OFF_SFT)
    oh |= onehot_t(et_ref, _OFF_ET)
    ohb = oh.astype(jnp.bfloat16)

    dn = (((0,), (0,)), ((), ()))
    aux = lax.dot_general(ohb, hi_ref[...], dn,
                          preferred_element_type=jnp.float32)
    aux += lax.dot_general(ohb, lo_ref[...], dn,
                           preferred_element_type=jnp.float32)
    x = x + aux

    mu = jnp.mean(x, axis=-1, keepdims=True)
    xc = x - mu
    var = jnp.mean(xc * xc, axis=-1, keepdims=True)
    y = xc * lax.rsqrt(var + EPS)
    out_ref[...] = y * gamma_ref[...] + beta_ref[...]


def _tc_call(gathered, pos_emb, idxs, small_hi, small_lo, gamma2d, beta2d):
    tok = lambda sb, b: b * _SBLK + sb
    idx_spec = pl.BlockSpec((1, 1, _T), lambda sb, b: (tok(sb, b), 0, 0))
    return pl.pallas_call(
        _tc_body,
        grid=(_SBLK, _NBLK // _SBLK),
        in_specs=[
            pl.BlockSpec((_T, HIDDEN), lambda sb, b: (tok(sb, b), 0)),
            pl.BlockSpec((_T, HIDDEN), lambda sb, b: (sb, 0)),
            idx_spec, idx_spec, idx_spec, idx_spec, idx_spec, idx_spec,
            pl.BlockSpec((_NSMALL, HIDDEN), lambda sb, b: (0, 0)),
            pl.BlockSpec((_NSMALL, HIDDEN), lambda sb, b: (0, 0)),
            pl.BlockSpec((1, HIDDEN), lambda sb, b: (0, 0)),
            pl.BlockSpec((1, HIDDEN), lambda sb, b: (0, 0)),
        ],
        out_specs=pl.BlockSpec((_T, HIDDEN), lambda sb, b: (tok(sb, b), 0)),
        out_shape=jax.ShapeDtypeStruct((NTOK, HIDDEN), jnp.float32),
    )(gathered, pos_emb, *idxs, small_hi, small_lo, gamma2d, beta2d)


def kernel(input_ids, token_type_ids, match_entity, sf_entity, match_token,
           sf_token, etype_ids, word_emb, token_type_emb, pos_emb,
           match_entity_emb, sf_entity_emb, match_token_emb, sf_token_emb,
           etype_emb, gamma, beta):
    ids = input_ids.reshape(NTOK).astype(jnp.int32)
    gathered = _make_sc_gather()(ids, word_emb)

    def prep(a):
        return a.reshape(_NBLK, 1, _T).astype(jnp.int32)

    idxs = (prep(token_type_ids), prep(match_entity), prep(match_token),
            prep(sf_entity), prep(sf_token), prep(etype_ids))

    small = jnp.zeros((_NSMALL, HIDDEN), jnp.float32)
    small = small.at[_OFF_TT:_OFF_TT + 2].set(token_type_emb)
    small = small.at[_OFF_ME:_OFF_ME + 2].set(match_entity_emb)
    small = small.at[_OFF_MT:_OFF_MT + 2].set(match_token_emb)
    small = small.at[_OFF_SFE:_OFF_SFE + SF_LEVEL].set(sf_entity_emb)
    small = small.at[_OFF_SFT:_OFF_SFT + SF_LEVEL].set(sf_token_emb)
    small = small.at[_OFF_ET:_OFF_ET + N_ETYPE].set(etype_emb)
    small_hi = small.astype(jnp.bfloat16)
    small_lo = (small - small_hi.astype(jnp.float32)).astype(jnp.bfloat16)

    out = _tc_call(gathered, pos_emb, idxs, small_hi, small_lo,
                   gamma.reshape(1, HIDDEN), beta.reshape(1, HIDDEN))
    return out.reshape(B, S, HIDDEN)


# speedup vs baseline: 5.3121x; 1.0009x over previous
import functools

import jax
import jax.numpy as jnp
from jax import lax
from jax.experimental import pallas as pl
from jax.experimental.pallas import tpu as pltpu
from jax.experimental.pallas import tpu_sc as plsc

VOCAB = 30522
HIDDEN = 768
MAX_POS = 2048
SF_LEVEL = 8
N_ETYPE = 16
B, S = 4, 2048
EPS = 1e-12

NTOK = B * S


_NC = 2
_NS = 16
_NW = _NC * _NS
_ROWS_PER_W = NTOK // _NW
_CHUNK = 64
_NCH = _ROWS_PER_W // _CHUNK


@functools.cache
def _make_sc_gather():
    mesh = plsc.VectorSubcoreMesh(core_axis_name="c", subcore_axis_name="s")

    @functools.partial(
        pl.kernel,
        mesh=mesh,
        out_type=jax.ShapeDtypeStruct((NTOK, HIDDEN), jnp.float32),
        scratch_types=[
            pltpu.VMEM((_ROWS_PER_W,), jnp.int32),
            pltpu.VMEM((2, _CHUNK, HIDDEN), jnp.float32),
            pltpu.SemaphoreType.DMA,
            pltpu.SemaphoreType.DMA,
            pltpu.SemaphoreType.DMA,
            pltpu.SemaphoreType.DMA,
        ],
    )
    def _sc_gather(ids_hbm, table_hbm, out_hbm, idx_v, rows_v, g0, g1, w0, w1):
        wid = lax.axis_index("s") * _NC + lax.axis_index("c")
        base = wid * _ROWS_PER_W
        pltpu.sync_copy(ids_hbm.at[pl.ds(base, _ROWS_PER_W)], idx_v)

        gsems = (g0, g1)
        wsems = (w0, w1)
        gathers = [None] * _NCH
        writes = [None] * _NCH

        def _issue_gather(ci):
            return pltpu.async_copy(
                table_hbm.at[idx_v.at[pl.ds(ci * _CHUNK, _CHUNK)]],
                rows_v.at[ci % 2],
                gsems[ci % 2],
            )

        gathers[0] = _issue_gather(0)
        for ci in range(_NCH):
            if ci + 1 < _NCH:
                if ci - 1 >= 0:
                    writes[ci - 1].wait()
                gathers[ci + 1] = _issue_gather(ci + 1)
            gathers[ci].wait()
            writes[ci] = pltpu.async_copy(
                rows_v.at[ci % 2],
                out_hbm.at[pl.ds(base + ci * _CHUNK, _CHUNK)],
                wsems[ci % 2],
            )
        writes[_NCH - 2].wait()
        writes[_NCH - 1].wait()

    return _sc_gather


_T = 512
_NBLK = NTOK // _T
_SBLK = S // _T
_NSMALL = 64

_OFF_TT = 0
_OFF_ME = 2
_OFF_MT = 4
_OFF_SFE = 6
_OFF_SFT = 14
_OFF_ET = 22


def _tc_body(g_ref, p_ref, tt_ref, me_ref, mt_ref, sfe_ref, sft_ref, et_ref,
             small_ref, gamma_ref, beta_ref, out_ref):
    x = g_ref[...] + p_ref[...]

    row = lax.broadcasted_iota(jnp.int32, (_NSMALL, _T), 0)

    def onehot_t(idx_ref, off):
        idx = idx_ref[0, :, :]
        return row == idx + off

    tt = (tt_ref[0, :, :] > 0).astype(jnp.int32)
    oh = (row == tt + _OFF_TT)
    oh |= onehot_t(me_ref, _OFF_ME)
    oh |= onehot_t(mt_ref, _OFF_MT)
    oh |= onehot_t(sfe_ref, _OFF_SFE)
    oh |= onehot_t(sft_ref, _OFF_SFT)
    oh |= onehot_t(et_ref, _OFF_ET)
    ohb = oh.astype(jnp.bfloat16)

    small = small_ref[...]
    hi = small.astype(jnp.bfloat16)
    lo = (small - hi.astype(jnp.float32)).astype(jnp.bfloat16)
    dn = (((0,), (0,)), ((), ()))
    aux = lax.dot_general(ohb, hi, dn, preferred_element_type=jnp.float32)
    aux += lax.dot_general(ohb, lo, dn, preferred_element_type=jnp.float32)
    x = x + aux

    mu = jnp.mean(x, axis=-1, keepdims=True)
    xc = x - mu
    var = jnp.mean(xc * xc, axis=-1, keepdims=True)
    y = xc * lax.rsqrt(var + EPS)
    out_ref[...] = y * gamma_ref[...] + beta_ref[...]


def _tc_call(gathered, pos_emb, idxs, small, gamma2d, beta2d):
    tok = lambda sb, b: b * _SBLK + sb
    idx_spec = pl.BlockSpec((1, 1, _T), lambda sb, b: (tok(sb, b), 0, 0))
    return pl.pallas_call(
        _tc_body,
        grid=(_SBLK, _NBLK // _SBLK),
        in_specs=[
            pl.BlockSpec((_T, HIDDEN), lambda sb, b: (tok(sb, b), 0)),
            pl.BlockSpec((_T, HIDDEN), lambda sb, b: (sb, 0)),
            idx_spec, idx_spec, idx_spec, idx_spec, idx_spec, idx_spec,
            pl.BlockSpec((_NSMALL, HIDDEN), lambda sb, b: (0, 0)),
            pl.BlockSpec((1, HIDDEN), lambda sb, b: (0, 0)),
            pl.BlockSpec((1, HIDDEN), lambda sb, b: (0, 0)),
        ],
        out_specs=pl.BlockSpec((_T, HIDDEN), lambda sb, b: (tok(sb, b), 0)),
        out_shape=jax.ShapeDtypeStruct((NTOK, HIDDEN), jnp.float32),
    )(gathered, pos_emb, *idxs, small, gamma2d, beta2d)


def kernel(input_ids, token_type_ids, match_entity, sf_entity, match_token,
           sf_token, etype_ids, word_emb, token_type_emb, pos_emb,
           match_entity_emb, sf_entity_emb, match_token_emb, sf_token_emb,
           etype_emb, gamma, beta):
    ids = input_ids.reshape(NTOK).astype(jnp.int32)
    gathered = _make_sc_gather()(ids, word_emb)

    def prep(a):
        return a.reshape(_NBLK, 1, _T).astype(jnp.int32)

    idxs = (prep(token_type_ids), prep(match_entity), prep(match_token),
            prep(sf_entity), prep(sf_token), prep(etype_ids))

    small = jnp.zeros((_NSMALL, HIDDEN), jnp.float32)
    small = small.at[_OFF_TT:_OFF_TT + 2].set(token_type_emb)
    small = small.at[_OFF_ME:_OFF_ME + 2].set(match_entity_emb)
    small = small.at[_OFF_MT:_OFF_MT + 2].set(match_token_emb)
    small = small.at[_OFF_SFE:_OFF_SFE + SF_LEVEL].set(sf_entity_emb)
    small = small.at[_OFF_SFT:_OFF_SFT + SF_LEVEL].set(sf_token_emb)
    small = small.at[_OFF_ET:_OFF_ET + N_ETYPE].set(etype_emb)

    out = _tc_call(gathered, pos_emb, idxs, small,
                   gamma.reshape(1, HIDDEN), beta.reshape(1, HIDDEN))
    return out.reshape(B, S, HIDDEN)


# speedup vs baseline: 5.5565x; 1.0460x over previous
import functools

import jax
import jax.numpy as jnp
from jax import lax
from jax.experimental import pallas as pl
from jax.experimental.pallas import tpu as pltpu
from jax.experimental.pallas import tpu_sc as plsc

VOCAB = 30522
HIDDEN = 768
MAX_POS = 2048
SF_LEVEL = 8
N_ETYPE = 16
B, S = 4, 2048
EPS = 1e-12

NTOK = B * S


_NC = 2
_NS = 16
_NW = _NC * _NS
_ROWS_PER_W = NTOK // _NW
_CHUNK = 64
_NCH = _ROWS_PER_W // _CHUNK


@functools.cache
def _make_sc_gather():
    mesh = plsc.VectorSubcoreMesh(core_axis_name="c", subcore_axis_name="s")

    @functools.partial(
        pl.kernel,
        mesh=mesh,
        out_type=jax.ShapeDtypeStruct((NTOK, HIDDEN), jnp.float32),
        scratch_types=[
            pltpu.VMEM((_ROWS_PER_W,), jnp.int32),
            pltpu.VMEM((2, _CHUNK, HIDDEN), jnp.float32),
            pltpu.SemaphoreType.DMA,
            pltpu.SemaphoreType.DMA,
            pltpu.SemaphoreType.DMA,
            pltpu.SemaphoreType.DMA,
        ],
    )
    def _sc_gather(ids_hbm, table_hbm, out_hbm, idx_v, rows_v, g0, g1, w0, w1):
        wid = lax.axis_index("s") * _NC + lax.axis_index("c")
        base = wid * _ROWS_PER_W
        pltpu.sync_copy(ids_hbm.at[pl.ds(base, _ROWS_PER_W)], idx_v)

        gsems = (g0, g1)
        wsems = (w0, w1)
        gathers = [None] * _NCH
        writes = [None] * _NCH

        def _issue_gather(ci):
            return pltpu.async_copy(
                table_hbm.at[idx_v.at[pl.ds(ci * _CHUNK, _CHUNK)]],
                rows_v.at[ci % 2],
                gsems[ci % 2],
            )

        gathers[0] = _issue_gather(0)
        for ci in range(_NCH):
            if ci + 1 < _NCH:
                if ci - 1 >= 0:
                    writes[ci - 1].wait()
                gathers[ci + 1] = _issue_gather(ci + 1)
            gathers[ci].wait()
            writes[ci] = pltpu.async_copy(
                rows_v.at[ci % 2],
                out_hbm.at[pl.ds(base + ci * _CHUNK, _CHUNK)],
                wsems[ci % 2],
            )
        writes[_NCH - 2].wait()
        writes[_NCH - 1].wait()

    return _sc_gather


_T = 1024
_NBLK = NTOK // _T
_SBLK = S // _T
_NSMALL = 64

_OFF_TT = 0
_OFF_ME = 2
_OFF_MT = 4
_OFF_SFE = 6
_OFF_SFT = 14
_OFF_ET = 22


def _tc_body(g_ref, p_ref, tt_ref, me_ref, mt_ref, sfe_ref, sft_ref, et_ref,
             small_ref, gamma_ref, beta_ref, out_ref):
    x = g_ref[...] + p_ref[...]

    row = lax.broadcasted_iota(jnp.int32, (_NSMALL, _T), 0)

    def onehot_t(idx_ref, off):
        idx = idx_ref[0, :, :]
        return row == idx + off

    tt = (tt_ref[0, :, :] > 0).astype(jnp.int32)
    oh = (row == tt + _OFF_TT)
    oh |= onehot_t(me_ref, _OFF_ME)
    oh |= onehot_t(mt_ref, _OFF_MT)
    oh |= onehot_t(sfe_ref, _OFF_SFE)
    oh |= onehot_t(sft_ref, _OFF_SFT)
    oh |= onehot_t(et_ref, _OFF_ET)
    ohb = oh.astype(jnp.bfloat16)

    small = small_ref[...]
    hi = small.astype(jnp.bfloat16)
    lo = (small - hi.astype(jnp.float32)).astype(jnp.bfloat16)
    dn = (((0,), (0,)), ((), ()))
    aux = lax.dot_general(ohb, hi, dn, preferred_element_type=jnp.float32)
    aux += lax.dot_general(ohb, lo, dn, preferred_element_type=jnp.float32)
    x = x + aux

    mu = jnp.mean(x, axis=-1, keepdims=True)
    xc = x - mu
    var = jnp.mean(xc * xc, axis=-1, keepdims=True)
    y = xc * lax.rsqrt(var + EPS)
    out_ref[...] = y * gamma_ref[...] + beta_ref[...]


def _tc_call(gathered, pos_emb, idxs, small, gamma2d, beta2d):
    tok = lambda sb, b: b * _SBLK + sb
    idx_spec = pl.BlockSpec((1, 1, _T), lambda sb, b: (tok(sb, b), 0, 0))
    return pl.pallas_call(
        _tc_body,
        grid=(_SBLK, _NBLK // _SBLK),
        in_specs=[
            pl.BlockSpec((_T, HIDDEN), lambda sb, b: (tok(sb, b), 0)),
            pl.BlockSpec((_T, HIDDEN), lambda sb, b: (sb, 0)),
            idx_spec, idx_spec, idx_spec, idx_spec, idx_spec, idx_spec,
            pl.BlockSpec((_NSMALL, HIDDEN), lambda sb, b: (0, 0)),
            pl.BlockSpec((1, HIDDEN), lambda sb, b: (0, 0)),
            pl.BlockSpec((1, HIDDEN), lambda sb, b: (0, 0)),
        ],
        out_specs=pl.BlockSpec((_T, HIDDEN), lambda sb, b: (tok(sb, b), 0)),
        out_shape=jax.ShapeDtypeStruct((NTOK, HIDDEN), jnp.float32),
    )(gathered, pos_emb, *idxs, small, gamma2d, beta2d)


def kernel(input_ids, token_type_ids, match_entity, sf_entity, match_token,
           sf_token, etype_ids, word_emb, token_type_emb, pos_emb,
           match_entity_emb, sf_entity_emb, match_token_emb, sf_token_emb,
           etype_emb, gamma, beta):
    ids = input_ids.reshape(NTOK).astype(jnp.int32)
    gathered = _make_sc_gather()(ids, word_emb)

    def prep(a):
        return a.reshape(_NBLK, 1, _T).astype(jnp.int32)

    idxs = (prep(token_type_ids), prep(match_entity), prep(match_token),
            prep(sf_entity), prep(sf_token), prep(etype_ids))

    small = jnp.zeros((_NSMALL, HIDDEN), jnp.float32)
    small = small.at[_OFF_TT:_OFF_TT + 2].set(token_type_emb)
    small = small.at[_OFF_ME:_OFF_ME + 2].set(match_entity_emb)
    small = small.at[_OFF_MT:_OFF_MT + 2].set(match_token_emb)
    small = small.at[_OFF_SFE:_OFF_SFE + SF_LEVEL].set(sf_entity_emb)
    small = small.at[_OFF_SFT:_OFF_SFT + SF_LEVEL].set(sf_token_emb)
    small = small.at[_OFF_ET:_OFF_ET + N_ETYPE].set(etype_emb)

    out = _tc_call(gathered, pos_emb, idxs, small,
                   gamma.reshape(1, HIDDEN), beta.reshape(1, HIDDEN))
    return out.reshape(B, S, HIDDEN)


# speedup vs baseline: 5.6155x; 1.0106x over previous
import functools

import jax
import jax.numpy as jnp
from jax import lax
from jax.experimental import pallas as pl
from jax.experimental.pallas import tpu as pltpu
from jax.experimental.pallas import tpu_sc as plsc

VOCAB = 30522
HIDDEN = 768
MAX_POS = 2048
SF_LEVEL = 8
N_ETYPE = 16
B, S = 4, 2048
EPS = 1e-12

NTOK = B * S


_NC = 2
_NS = 16
_NW = _NC * _NS
_ROWS_PER_W = NTOK // _NW
_CHUNK = 64
_NCH = _ROWS_PER_W // _CHUNK


@functools.cache
def _make_sc_gather():
    mesh = plsc.VectorSubcoreMesh(core_axis_name="c", subcore_axis_name="s")

    @functools.partial(
        pl.kernel,
        mesh=mesh,
        out_type=jax.ShapeDtypeStruct((NTOK, HIDDEN), jnp.float32),
        scratch_types=[
            pltpu.VMEM((_ROWS_PER_W,), jnp.int32),
            pltpu.VMEM((2, _CHUNK, HIDDEN), jnp.float32),
            pltpu.SemaphoreType.DMA,
            pltpu.SemaphoreType.DMA,
            pltpu.SemaphoreType.DMA,
            pltpu.SemaphoreType.DMA,
        ],
    )
    def _sc_gather(ids_hbm, table_hbm, out_hbm, idx_v, rows_v, g0, g1, w0, w1):
        wid = lax.axis_index("s") * _NC + lax.axis_index("c")
        base = wid * _ROWS_PER_W
        pltpu.sync_copy(ids_hbm.at[pl.ds(base, _ROWS_PER_W)], idx_v)

        gsems = (g0, g1)
        wsems = (w0, w1)
        gathers = [None] * _NCH
        writes = [None] * _NCH

        def _issue_gather(ci):
            return pltpu.async_copy(
                table_hbm.at[idx_v.at[pl.ds(ci * _CHUNK, _CHUNK)]],
                rows_v.at[ci % 2],
                gsems[ci % 2],
            )

        gathers[0] = _issue_gather(0)
        for ci in range(_NCH):
            if ci + 1 < _NCH:
                if ci - 1 >= 0:
                    writes[ci - 1].wait()
                gathers[ci + 1] = _issue_gather(ci + 1)
            gathers[ci].wait()
            writes[ci] = pltpu.async_copy(
                rows_v.at[ci % 2],
                out_hbm.at[pl.ds(base + ci * _CHUNK, _CHUNK)],
                wsems[ci % 2],
            )
        writes[_NCH - 2].wait()
        writes[_NCH - 1].wait()

    return _sc_gather


_T = 2048
_NBLK = NTOK // _T
_SBLK = S // _T
_NSMALL = 64

_OFF_TT = 0
_OFF_ME = 2
_OFF_MT = 4
_OFF_SFE = 6
_OFF_SFT = 14
_OFF_ET = 22


def _tc_body(g_ref, p_ref, tt_ref, me_ref, mt_ref, sfe_ref, sft_ref, et_ref,
             small_ref, gamma_ref, beta_ref, out_ref):
    x = g_ref[...] + p_ref[...]

    row = lax.broadcasted_iota(jnp.int32, (_NSMALL, _T), 0)

    def onehot_t(idx_ref, off):
        idx = idx_ref[0, :, :]
        return row == idx + off

    tt = (tt_ref[0, :, :] > 0).astype(jnp.int32)
    oh = (row == tt + _OFF_TT)
    oh |= onehot_t(me_ref, _OFF_ME)
    oh |= onehot_t(mt_ref, _OFF_MT)
    oh |= onehot_t(sfe_ref, _OFF_SFE)
    oh |= onehot_t(sft_ref, _OFF_SFT)
    oh |= onehot_t(et_ref, _OFF_ET)
    ohb = oh.astype(jnp.bfloat16)

    small = small_ref[...]
    hi = small.astype(jnp.bfloat16)
    lo = (small - hi.astype(jnp.float32)).astype(jnp.bfloat16)
    dn = (((0,), (0,)), ((), ()))
    aux = lax.dot_general(ohb, hi, dn, preferred_element_type=jnp.float32)
    aux += lax.dot_general(ohb, lo, dn, preferred_element_type=jnp.float32)
    x = x + aux

    mu = jnp.mean(x, axis=-1, keepdims=True)
    xc = x - mu
    var = jnp.mean(xc * xc, axis=-1, keepdims=True)
    y = xc * lax.rsqrt(var + EPS)
    out_ref[...] = y * gamma_ref[...] + beta_ref[...]


def _tc_call(gathered, pos_emb, idxs, small, gamma2d, beta2d):
    tok = lambda sb, b: b * _SBLK + sb
    idx_spec = pl.BlockSpec((1, 1, _T), lambda sb, b: (tok(sb, b), 0, 0))
    return pl.pallas_call(
        _tc_body,
        grid=(_SBLK, _NBLK // _SBLK),
        in_specs=[
            pl.BlockSpec((_T, HIDDEN), lambda sb, b: (tok(sb, b), 0)),
            pl.BlockSpec((_T, HIDDEN), lambda sb, b: (sb, 0)),
            idx_spec, idx_spec, idx_spec, idx_spec, idx_spec, idx_spec,
            pl.BlockSpec((_NSMALL, HIDDEN), lambda sb, b: (0, 0)),
            pl.BlockSpec((1, HIDDEN), lambda sb, b: (0, 0)),
            pl.BlockSpec((1, HIDDEN), lambda sb, b: (0, 0)),
        ],
        out_specs=pl.BlockSpec((_T, HIDDEN), lambda sb, b: (tok(sb, b), 0)),
        out_shape=jax.ShapeDtypeStruct((NTOK, HIDDEN), jnp.float32),
    )(gathered, pos_emb, *idxs, small, gamma2d, beta2d)


def kernel(input_ids, token_type_ids, match_entity, sf_entity, match_token,
           sf_token, etype_ids, word_emb, token_type_emb, pos_emb,
           match_entity_emb, sf_entity_emb, match_token_emb, sf_token_emb,
           etype_emb, gamma, beta):
    ids = input_ids.reshape(NTOK).astype(jnp.int32)
    gathered = _make_sc_gather()(ids, word_emb)

    def prep(a):
        return a.reshape(_NBLK, 1, _T).astype(jnp.int32)

    idxs = (prep(token_type_ids), prep(match_entity), prep(match_token),
            prep(sf_entity), prep(sf_token), prep(etype_ids))

    small = jnp.zeros((_NSMALL, HIDDEN), jnp.float32)
    small = small.at[_OFF_TT:_OFF_TT + 2].set(token_type_emb)
    small = small.at[_OFF_ME:_OFF_ME + 2].set(match_entity_emb)
    small = small.at[_OFF_MT:_OFF_MT + 2].set(match_token_emb)
    small = small.at[_OFF_SFE:_OFF_SFE + SF_LEVEL].set(sf_entity_emb)
    small = small.at[_OFF_SFT:_OFF_SFT + SF_LEVEL].set(sf_token_emb)
    small = small.at[_OFF_ET:_OFF_ET + N_ETYPE].set(etype_emb)

    out = _tc_call(gathered, pos_emb, idxs, small,
                   gamma.reshape(1, HIDDEN), beta.reshape(1, HIDDEN))
    return out.reshape(B, S, HIDDEN)
